# baseline (speedup 1.0000x reference)
import jax
import jax.numpy as jnp
from jax import lax
from jax.experimental import pallas as pl
from jax.experimental.pallas import tpu as pltpu
from jax.experimental.pallas import tpu_sc as plsc

N_NODES = 50000
N_EDGES = 1600000
DT = 0.01
ITERATION = 10

NC = 2
NS = 16
NW = NC * NS
NPAD = 50176
ROWS_T = NPAD // NW
ROWS_C = NPAD // NS
CH = 2048
NCH = 25
EPW = NCH * CH
EPAD = EPW * NW

_MAGIC = 0x5F3759DF

_CPARAMS = None


def _cparams():
    global _CPARAMS
    if _CPARAMS is None:
        _CPARAMS = pltpu.CompilerParams(needs_layout_passes=False,
                                        use_tc_tiling_on_sc=False)
    return _CPARAMS


def _mesh():
    return plsc.VectorSubcoreMesh(core_axis_name="c", subcore_axis_name="s",
                                  num_cores=NC)


def _edge_chunks(body_fn):
    @pl.loop(0, NCH)
    def _c(c):
        body_fn(c)


def _prep_body(x_h, y_h, z_h, vx_h, vy_h, vz_h, fx_h, fy_h, fz_h, wn_h, cn_h,
               i1_h, j1_h,
               px_h, py_h, pz_h, e0x_h, e0y_h, e0z_h, e1x_h, e1y_h, e1z_h,
               k_h, ak_h, wi_h, wj_h, l_h,
               ii_v, jj_v, k_v, ak_v, wi_v, wj_v, tmp_v, zc_v,
               buf_a, buf_b, buf_c, buf_w):
    cid = lax.axis_index("c")
    sid = lax.axis_index("s")
    wid = cid * NS + sid
    f32 = jnp.float32
    dt = f32(DT)
    dt2 = f32(DT * DT)
    zero16 = jnp.zeros((16,), f32)

    tsl = pl.ds(wid * ROWS_T, ROWS_T)

    pltpu.sync_copy(wn_h.at[tsl], buf_w)
    for pos_h, vel_h, f_h, p_h in ((x_h, vx_h, fx_h, px_h),
                                   (y_h, vy_h, fy_h, py_h),
                                   (z_h, vz_h, fz_h, pz_h)):
        pltpu.sync_copy(pos_h.at[tsl], buf_a)
        pltpu.sync_copy(vel_h.at[tsl], buf_b)
        pltpu.sync_copy(f_h.at[tsl], buf_c)

        @pl.loop(0, ROWS_T // 16)
        def _pred(t):
            s = pl.ds(t * 16, 16)
            buf_a[s] = buf_a[s] + dt * buf_b[s] + dt2 * buf_w[s] * buf_c[s]

        pltpu.sync_copy(buf_a, p_h.at[tsl])

    @pl.loop(0, ROWS_T // 16)
    def _zb(t):
        buf_a[pl.ds(t * 16, 16)] = zero16

    for d_h in (e0x_h, e0y_h, e0z_h, e1x_h, e1y_h, e1z_h):
        pltpu.sync_copy(buf_a, d_h.at[tsl])

    @pl.loop(0, CH // 16)
    def _zc(t):
        zc_v[pl.ds(t * 16, 16)] = zero16

    @pl.loop(0, NCH)
    def _coef(c):
        sl = pl.ds(wid * EPW + c * CH, CH)
        pltpu.sync_copy(i1_h.at[sl], ii_v)
        pltpu.sync_copy(j1_h.at[sl], jj_v)
        pltpu.sync_copy(wn_h.at[ii_v], wi_v)
        pltpu.sync_copy(wn_h.at[jj_v], wj_v)
        pltpu.sync_copy(cn_h.at[ii_v], k_v)
        pltpu.sync_copy(cn_h.at[jj_v], tmp_v)

        @pl.loop(0, CH // 16)
        def _ck(t):
            s = pl.ds(t * 16, 16)
            wi = wi_v[s]
            wj = wj_v[s]
            a = f32(0.5) * (k_v[s] + tmp_v[s])
            ssum = wi + wj
            k = jnp.where(ssum == 0.0, f32(0.0), f32(1.0) / (ssum + a))
            k_v[s] = k
            ak_v[s] = a * k

        pltpu.sync_copy(k_v, k_h.at[sl])
        pltpu.sync_copy(ak_v, ak_h.at[sl])
        pltpu.sync_copy(wi_v, wi_h.at[sl])
        pltpu.sync_copy(wj_v, wj_h.at[sl])
        pltpu.sync_copy(zc_v, l_h.at[sl])


def _iter_body(pxp_h, pyp_h, pzp_h, d0x_h, d0y_h, d0z_h, d1x_h, d1y_h, d1z_h,
               i1_h, j1_h, k_h, ak_h, wi_h, wj_h, d0e_h, lp_h,
               px_h, py_h, pz_h, e0x_h, e0y_h, e0z_h, e1x_h, e1y_h, e1z_h,
               ln_h,
               ax_sh, ay_sh, az_sh,
               ii_v, jj_v, k_v, ak_v, wi_v, wj_v, d0_v, l_v,
               gxi, gyi, gzi, gxj, gyj, gzj,
               dxi, dyi, dzi, dxj, dyj, dzj,
               buf_a, buf_b, buf_c):
    cid = lax.axis_index("c")
    sid = lax.axis_index("s")
    wid = cid * NS + sid
    f32 = jnp.float32
    zero16 = jnp.zeros((16,), f32)
    nan16 = zero16 + f32(jnp.nan)

    csl = pl.ds(sid * ROWS_C, ROWS_C)

    for pp_h, da_h, db_h, pn_h in ((pxp_h, d0x_h, d1x_h, px_h),
                                   (pyp_h, d0y_h, d1y_h, py_h),
                                   (pzp_h, d0z_h, d1z_h, pz_h)):
        pltpu.sync_copy(pp_h.at[csl], buf_a)
        pltpu.sync_copy(da_h.at[csl], buf_b)
        pltpu.sync_copy(db_h.at[csl], buf_c)

        @pl.loop(0, ROWS_C // 16)
        def _cmb(t):
            s = pl.ds(t * 16, 16)
            buf_a[s] = buf_a[s] + buf_b[s] + buf_c[s]

        pltpu.sync_copy(buf_a, pn_h.at[csl])

    @pl.loop(0, ROWS_C // 16)
    def _zb(t):
        buf_b[pl.ds(t * 16, 16)] = zero16

    pltpu.sync_copy(buf_b, ax_sh.at[csl])
    pltpu.sync_copy(buf_b, ay_sh.at[csl])
    pltpu.sync_copy(buf_b, az_sh.at[csl])
    plsc.subcore_barrier()

    @pl.loop(0, NCH)
    def _chunk(c):
        sl = pl.ds(wid * EPW + c * CH, CH)
        pltpu.sync_copy(i1_h.at[sl], ii_v)
        pltpu.sync_copy(j1_h.at[sl], jj_v)
        pltpu.sync_copy(k_h.at[sl], k_v)
        pltpu.sync_copy(ak_h.at[sl], ak_v)
        pltpu.sync_copy(wi_h.at[sl], wi_v)
        pltpu.sync_copy(wj_h.at[sl], wj_v)
        pltpu.sync_copy(d0e_h.at[sl], d0_v)
        pltpu.sync_copy(lp_h.at[sl], l_v)
        pltpu.sync_copy(px_h.at[ii_v], gxi)
        pltpu.sync_copy(py_h.at[ii_v], gyi)
        pltpu.sync_copy(pz_h.at[ii_v], gzi)
        pltpu.sync_copy(px_h.at[jj_v], gxj)
        pltpu.sync_copy(py_h.at[jj_v], gyj)
        pltpu.sync_copy(pz_h.at[jj_v], gzj)

        @pl.loop(0, CH // 16)
        def _edge(t):
            s = pl.ds(t * 16, 16)
            dx = gxi[s] - gxj[s]
            dy = gyi[s] - gyj[s]
            dz = gzi[s] - gzj[s]
            d2 = dx * dx + dy * dy + dz * dz
            bits = plsc.bitcast(d2, jnp.int32)
            y = plsc.bitcast(_MAGIC - (bits >> 1), f32)
            hd = f32(0.5) * d2
            y = y * (f32(1.5) - hd * y * y)
            y = y * (f32(1.5) - hd * y * y)
            dnorm = d2 * y
            invd = jnp.where(d2 == 0.0, nan16, y)
            el = l_v[s]
            ld = (d0_v[s] - dnorm) * k_v[s] - ak_v[s] * el
            l_v[s] = el + ld
            g = ld * invd
            ai = wi_v[s] * g
            aj = -(wj_v[s] * g)
            dxi[s] = ai * dx
            dyi[s] = ai * dy
            dzi[s] = ai * dz
            dxj[s] = aj * dx
            dyj[s] = aj * dy
            dzj[s] = aj * dz

        pltpu.sync_copy(l_v, ln_h.at[sl])
        pltpu.sync_copy(dxi, ax_sh.at[ii_v], add=True)
        pltpu.sync_copy(dyi, ay_sh.at[ii_v], add=True)
        pltpu.sync_copy(dzi, az_sh.at[ii_v], add=True)
        pltpu.sync_copy(dxj, ax_sh.at[jj_v], add=True)
        pltpu.sync_copy(dyj, ay_sh.at[jj_v], add=True)
        pltpu.sync_copy(dzj, az_sh.at[jj_v], add=True)

    plsc.subcore_barrier()

    @pl.when(cid == 0)
    def _pub0():
        for acc_sh, e_h in ((ax_sh, e0x_h), (ay_sh, e0y_h), (az_sh, e0z_h)):
            pltpu.sync_copy(acc_sh.at[csl], buf_a)
            pltpu.sync_copy(buf_a, e_h.at[csl])

    @pl.when(cid == 1)
    def _pub1():
        for acc_sh, e_h in ((ax_sh, e1x_h), (ay_sh, e1y_h), (az_sh, e1z_h)):
            pltpu.sync_copy(acc_sh.at[csl], buf_a)
            pltpu.sync_copy(buf_a, e_h.at[csl])


def _final_body(pxp_h, pyp_h, pzp_h, d0x_h, d0y_h, d0z_h,
                d1x_h, d1y_h, d1z_h, x_h, y_h, z_h,
                px_h, py_h, pz_h, ux_h, uy_h, uz_h,
                buf_a, buf_b, buf_c):
    cid = lax.axis_index("c")
    sid = lax.axis_index("s")
    wid = cid * NS + sid
    f32 = jnp.float32
    dt = f32(DT)
    tsl = pl.ds(wid * ROWS_T, ROWS_T)

    for pp_h, da_h, db_h, o_h, pn_h, u_h in (
            (pxp_h, d0x_h, d1x_h, x_h, px_h, ux_h),
            (pyp_h, d0y_h, d1y_h, y_h, py_h, uy_h),
            (pzp_h, d0z_h, d1z_h, z_h, pz_h, uz_h)):
        pltpu.sync_copy(pp_h.at[tsl], buf_a)
        pltpu.sync_copy(da_h.at[tsl], buf_b)
        pltpu.sync_copy(db_h.at[tsl], buf_c)

        @pl.loop(0, ROWS_T // 16)
        def _cmb(t):
            s = pl.ds(t * 16, 16)
            buf_a[s] = buf_a[s] + buf_b[s] + buf_c[s]

        pltpu.sync_copy(buf_a, pn_h.at[tsl])
        pltpu.sync_copy(o_h.at[tsl], buf_b)

        @pl.loop(0, ROWS_T // 16)
        def _vel(t):
            s = pl.ds(t * 16, 16)
            buf_b[s] = (buf_a[s] - buf_b[s]) / dt

        pltpu.sync_copy(buf_b, u_h.at[tsl])


@jax.jit
def _xpbd(x, y, z, vx, vy, vz, fx, fy, fz, wn, cn, i1, j1, d0e):
    f32 = jnp.float32
    node = jax.ShapeDtypeStruct((NPAD,), f32)
    edge = jax.ShapeDtypeStruct((EPAD,), f32)
    evmem = pltpu.VMEM((CH,), f32)
    tvmem = pltpu.VMEM((ROWS_T,), f32)
    cvmem = pltpu.VMEM((ROWS_C,), f32)

    prep = pl.kernel(
        _prep_body,
        out_type=(node,) * 9 + (edge,) * 5,
        mesh=_mesh(),
        scratch_types=[pltpu.VMEM((CH,), jnp.int32),
                       pltpu.VMEM((CH,), jnp.int32),
                       evmem, evmem, evmem, evmem, evmem, evmem,
                       tvmem, tvmem, tvmem, tvmem],
        compiler_params=_cparams())
    px, py, pz, e0x, e0y, e0z, e1x, e1y, e1z, kk, ak, wi, wj, l = prep(
        x, y, z, vx, vy, vz, fx, fy, fz, wn, cn, i1, j1)

    itk = pl.kernel(
        _iter_body,
        out_type=(node,) * 9 + (edge,),
        mesh=_mesh(),
        scratch_types=[pltpu.VMEM_SHARED((NPAD,), f32),
                       pltpu.VMEM_SHARED((NPAD,), f32),
                       pltpu.VMEM_SHARED((NPAD,), f32),
                       pltpu.VMEM((CH,), jnp.int32),
                       pltpu.VMEM((CH,), jnp.int32),
                       evmem, evmem, evmem, evmem, evmem, evmem,
                       evmem, evmem, evmem, evmem, evmem, evmem,
                       evmem, evmem, evmem, evmem, evmem, evmem,
                       cvmem, cvmem, cvmem],
        compiler_params=_cparams())

    for _ in range(ITERATION):
        px, py, pz, e0x, e0y, e0z, e1x, e1y, e1z, l = itk(
            px, py, pz, e0x, e0y, e0z, e1x, e1y, e1z,
            i1, j1, kk, ak, wi, wj, d0e, l)

    fin = pl.kernel(
        _final_body,
        out_type=(node,) * 6,
        mesh=_mesh(),
        scratch_types=[tvmem, tvmem, tvmem],
        compiler_params=_cparams())
    return fin(px, py, pz, e0x, e0y, e0z, e1x, e1y, e1z, x, y, z)


def kernel(V, V_velocity, V_w, V_force, V_compliance, C_dist, C_init_d):
    f32 = jnp.float32
    n = V.shape[0]
    e = C_dist.shape[0]

    def padn(a):
        return jnp.zeros((NPAD,), f32).at[:n].set(a.astype(f32))

    V = V.astype(f32)
    x = padn(V[:, 0])
    x = x.at[n:].set(jnp.float32(1.0) + jnp.arange(NPAD - n, dtype=f32))
    y = padn(V[:, 1])
    z = padn(V[:, 2])
    vx = padn(V_velocity[:, 0])
    vy = padn(V_velocity[:, 1])
    vz = padn(V_velocity[:, 2])
    fx = padn(V_force[:, 0])
    fy = padn(V_force[:, 1])
    fz = padn(V_force[:, 2])
    wn = padn(V_w[:, 0])
    cn = padn(V_compliance[:, 0])
    i1 = jnp.full((EPAD,), n, jnp.int32).at[:e].set(C_dist[:, 0].astype(jnp.int32))
    j1 = jnp.full((EPAD,), n + 1, jnp.int32).at[:e].set(C_dist[:, 1].astype(jnp.int32))
    d0 = jnp.ones((EPAD,), f32).at[:e].set(C_init_d[:, 0].astype(f32))

    px, py, pz, ux, uy, uz = _xpbd(x, y, z, vx, vy, vz, fx, fy, fz,
                                   wn, cn, i1, j1, d0)
    Vout = jnp.stack([px[:n], py[:n], pz[:n]], axis=1)
    Velout = jnp.stack([ux[:n], uy[:n], uz[:n]], axis=1)
    return Vout, Velout


# speedup vs baseline: 2.3887x; 2.3887x over previous
import jax
import jax.numpy as jnp
from jax import lax
from jax.experimental import pallas as pl
from jax.experimental.pallas import tpu as pltpu
from jax.experimental.pallas import tpu_sc as plsc

N_NODES = 50000
N_EDGES = 1600000
DT = 0.01
ITERATION = 10

NC = 2
NS = 16
NW = NC * NS
NPAD = 50176
ROWS_T = NPAD // NW
ROWS_C = NPAD // NS
CH = 2048
NCH = 25
EPW = NCH * CH
EPAD = EPW * NW

_MAGIC = 0x5F3759DF

_CPARAMS = None


def _cparams():
    global _CPARAMS
    if _CPARAMS is None:
        _CPARAMS = pltpu.CompilerParams(needs_layout_passes=False,
                                        use_tc_tiling_on_sc=False)
    return _CPARAMS


def _mesh():
    return plsc.VectorSubcoreMesh(core_axis_name="c", subcore_axis_name="s",
                                  num_cores=NC)


def _edge_chunks(body_fn):
    @pl.loop(0, NCH)
    def _c(c):
        body_fn(c)


def _prep_body(x_h, y_h, z_h, vx_h, vy_h, vz_h, fx_h, fy_h, fz_h, wn_h, cn_h,
               i1_h, j1_h,
               px_h, py_h, pz_h, e0x_h, e0y_h, e0z_h, e1x_h, e1y_h, e1z_h,
               k_h, ak_h, wi_h, wj_h, l_h,
               ii_v, jj_v, k_v, ak_v, wi_v, wj_v, tmp_v, zc_v,
               buf_a, buf_b, buf_c, buf_w):
    cid = lax.axis_index("c")
    sid = lax.axis_index("s")
    wid = cid * NS + sid
    f32 = jnp.float32
    dt = f32(DT)
    dt2 = f32(DT * DT)
    zero16 = jnp.zeros((16,), f32)

    tsl = pl.ds(wid * ROWS_T, ROWS_T)

    pltpu.sync_copy(wn_h.at[tsl], buf_w)
    for pos_h, vel_h, f_h, p_h in ((x_h, vx_h, fx_h, px_h),
                                   (y_h, vy_h, fy_h, py_h),
                                   (z_h, vz_h, fz_h, pz_h)):
        pltpu.sync_copy(pos_h.at[tsl], buf_a)
        pltpu.sync_copy(vel_h.at[tsl], buf_b)
        pltpu.sync_copy(f_h.at[tsl], buf_c)

        @pl.loop(0, ROWS_T // 16)
        def _pred(t):
            s = pl.ds(t * 16, 16)
            buf_a[s] = buf_a[s] + dt * buf_b[s] + dt2 * buf_w[s] * buf_c[s]

        pltpu.sync_copy(buf_a, p_h.at[tsl])

    @pl.loop(0, ROWS_T // 16)
    def _zb(t):
        buf_a[pl.ds(t * 16, 16)] = zero16

    for d_h in (e0x_h, e0y_h, e0z_h, e1x_h, e1y_h, e1z_h):
        pltpu.sync_copy(buf_a, d_h.at[tsl])

    @pl.loop(0, CH // 16)
    def _zc(t):
        zc_v[pl.ds(t * 16, 16)] = zero16

    @pl.loop(0, NCH)
    def _coef(c):
        sl = pl.ds(wid * EPW + c * CH, CH)
        pltpu.sync_copy(i1_h.at[sl], ii_v)
        pltpu.sync_copy(j1_h.at[sl], jj_v)
        pltpu.sync_copy(wn_h.at[ii_v], wi_v)
        pltpu.sync_copy(wn_h.at[jj_v], wj_v)
        pltpu.sync_copy(cn_h.at[ii_v], k_v)
        pltpu.sync_copy(cn_h.at[jj_v], tmp_v)

        @pl.loop(0, CH // 16)
        def _ck(t):
            s = pl.ds(t * 16, 16)
            wi = wi_v[s]
            wj = wj_v[s]
            a = f32(0.5) * (k_v[s] + tmp_v[s])
            ssum = wi + wj
            k = jnp.where(ssum == 0.0, f32(0.0), f32(1.0) / (ssum + a))
            k_v[s] = k
            ak_v[s] = a * k

        pltpu.sync_copy(k_v, k_h.at[sl])
        pltpu.sync_copy(ak_v, ak_h.at[sl])
        pltpu.sync_copy(wi_v, wi_h.at[sl])
        pltpu.sync_copy(wj_v, wj_h.at[sl])
        pltpu.sync_copy(zc_v, l_h.at[sl])


def _iter_body(pxp_h, pyp_h, pzp_h, d0x_h, d0y_h, d0z_h, d1x_h, d1y_h, d1z_h,
               i1_h, j1_h, k_h, ak_h, wi_h, wj_h, d0e_h, lp_h,
               px_h, py_h, pz_h, e0x_h, e0y_h, e0z_h, e1x_h, e1y_h, e1z_h,
               ln_h,
               ax_sh, ay_sh, az_sh, px_sh, py_sh, pz_sh,
               ii_v, jj_v, k_v, ak_v, wi_v, wj_v, d0_v, l_v,
               gxi, gyi, gzi, gxj, gyj, gzj,
               dxi, dyi, dzi, dxj, dyj, dzj,
               buf_a, buf_b, buf_c):
    cid = lax.axis_index("c")
    sid = lax.axis_index("s")
    wid = cid * NS + sid
    f32 = jnp.float32
    zero16 = jnp.zeros((16,), f32)
    nan16 = zero16 + f32(jnp.nan)

    csl = pl.ds(sid * ROWS_C, ROWS_C)

    for pp_h, da_h, db_h, pn_h, p_sh in ((pxp_h, d0x_h, d1x_h, px_h, px_sh),
                                         (pyp_h, d0y_h, d1y_h, py_h, py_sh),
                                         (pzp_h, d0z_h, d1z_h, pz_h, pz_sh)):
        pltpu.sync_copy(pp_h.at[csl], buf_a)
        pltpu.sync_copy(da_h.at[csl], buf_b)
        pltpu.sync_copy(db_h.at[csl], buf_c)

        @pl.loop(0, ROWS_C // 16)
        def _cmb(t):
            s = pl.ds(t * 16, 16)
            buf_a[s] = buf_a[s] + buf_b[s] + buf_c[s]

        pltpu.sync_copy(buf_a, pn_h.at[csl])
        pltpu.sync_copy(buf_a, p_sh.at[csl])

    @pl.loop(0, ROWS_C // 16)
    def _zb(t):
        buf_b[pl.ds(t * 16, 16)] = zero16

    pltpu.sync_copy(buf_b, ax_sh.at[csl])
    pltpu.sync_copy(buf_b, ay_sh.at[csl])
    pltpu.sync_copy(buf_b, az_sh.at[csl])
    plsc.subcore_barrier()

    @pl.loop(0, NCH)
    def _chunk(c):
        sl = pl.ds(wid * EPW + c * CH, CH)
        pltpu.sync_copy(i1_h.at[sl], ii_v)
        pltpu.sync_copy(j1_h.at[sl], jj_v)
        pltpu.sync_copy(k_h.at[sl], k_v)
        pltpu.sync_copy(ak_h.at[sl], ak_v)
        pltpu.sync_copy(wi_h.at[sl], wi_v)
        pltpu.sync_copy(wj_h.at[sl], wj_v)
        pltpu.sync_copy(d0e_h.at[sl], d0_v)
        pltpu.sync_copy(lp_h.at[sl], l_v)
        pltpu.sync_copy(px_sh.at[ii_v], gxi)
        pltpu.sync_copy(py_sh.at[ii_v], gyi)
        pltpu.sync_copy(pz_sh.at[ii_v], gzi)
        pltpu.sync_copy(px_sh.at[jj_v], gxj)
        pltpu.sync_copy(py_sh.at[jj_v], gyj)
        pltpu.sync_copy(pz_sh.at[jj_v], gzj)

        @pl.loop(0, CH // 16)
        def _edge(t):
            s = pl.ds(t * 16, 16)
            dx = gxi[s] - gxj[s]
            dy = gyi[s] - gyj[s]
            dz = gzi[s] - gzj[s]
            d2 = dx * dx + dy * dy + dz * dz
            bits = plsc.bitcast(d2, jnp.int32)
            y = plsc.bitcast(_MAGIC - (bits >> 1), f32)
            hd = f32(0.5) * d2
            y = y * (f32(1.5) - hd * y * y)
            y = y * (f32(1.5) - hd * y * y)
            dnorm = d2 * y
            invd = jnp.where(d2 == 0.0, nan16, y)
            el = l_v[s]
            ld = (d0_v[s] - dnorm) * k_v[s] - ak_v[s] * el
            l_v[s] = el + ld
            g = ld * invd
            ai = wi_v[s] * g
            aj = -(wj_v[s] * g)
            dxi[s] = ai * dx
            dyi[s] = ai * dy
            dzi[s] = ai * dz
            dxj[s] = aj * dx
            dyj[s] = aj * dy
            dzj[s] = aj * dz

        pltpu.sync_copy(l_v, ln_h.at[sl])
        pltpu.sync_copy(dxi, ax_sh.at[ii_v], add=True)
        pltpu.sync_copy(dyi, ay_sh.at[ii_v], add=True)
        pltpu.sync_copy(dzi, az_sh.at[ii_v], add=True)
        pltpu.sync_copy(dxj, ax_sh.at[jj_v], add=True)
        pltpu.sync_copy(dyj, ay_sh.at[jj_v], add=True)
        pltpu.sync_copy(dzj, az_sh.at[jj_v], add=True)

    plsc.subcore_barrier()

    @pl.when(cid == 0)
    def _pub0():
        for acc_sh, e_h in ((ax_sh, e0x_h), (ay_sh, e0y_h), (az_sh, e0z_h)):
            pltpu.sync_copy(acc_sh.at[csl], buf_a)
            pltpu.sync_copy(buf_a, e_h.at[csl])

    @pl.when(cid == 1)
    def _pub1():
        for acc_sh, e_h in ((ax_sh, e1x_h), (ay_sh, e1y_h), (az_sh, e1z_h)):
            pltpu.sync_copy(acc_sh.at[csl], buf_a)
            pltpu.sync_copy(buf_a, e_h.at[csl])


def _final_body(pxp_h, pyp_h, pzp_h, d0x_h, d0y_h, d0z_h,
                d1x_h, d1y_h, d1z_h, x_h, y_h, z_h,
                px_h, py_h, pz_h, ux_h, uy_h, uz_h,
                buf_a, buf_b, buf_c):
    cid = lax.axis_index("c")
    sid = lax.axis_index("s")
    wid = cid * NS + sid
    f32 = jnp.float32
    dt = f32(DT)
    tsl = pl.ds(wid * ROWS_T, ROWS_T)

    for pp_h, da_h, db_h, o_h, pn_h, u_h in (
            (pxp_h, d0x_h, d1x_h, x_h, px_h, ux_h),
            (pyp_h, d0y_h, d1y_h, y_h, py_h, uy_h),
            (pzp_h, d0z_h, d1z_h, z_h, pz_h, uz_h)):
        pltpu.sync_copy(pp_h.at[tsl], buf_a)
        pltpu.sync_copy(da_h.at[tsl], buf_b)
        pltpu.sync_copy(db_h.at[tsl], buf_c)

        @pl.loop(0, ROWS_T // 16)
        def _cmb(t):
            s = pl.ds(t * 16, 16)
            buf_a[s] = buf_a[s] + buf_b[s] + buf_c[s]

        pltpu.sync_copy(buf_a, pn_h.at[tsl])
        pltpu.sync_copy(o_h.at[tsl], buf_b)

        @pl.loop(0, ROWS_T // 16)
        def _vel(t):
            s = pl.ds(t * 16, 16)
            buf_b[s] = (buf_a[s] - buf_b[s]) / dt

        pltpu.sync_copy(buf_b, u_h.at[tsl])


@jax.jit
def _xpbd(x, y, z, vx, vy, vz, fx, fy, fz, wn, cn, i1, j1, d0e):
    f32 = jnp.float32
    node = jax.ShapeDtypeStruct((NPAD,), f32)
    edge = jax.ShapeDtypeStruct((EPAD,), f32)
    evmem = pltpu.VMEM((CH,), f32)
    tvmem = pltpu.VMEM((ROWS_T,), f32)
    cvmem = pltpu.VMEM((ROWS_C,), f32)

    prep = pl.kernel(
        _prep_body,
        out_type=(node,) * 9 + (edge,) * 5,
        mesh=_mesh(),
        scratch_types=[pltpu.VMEM((CH,), jnp.int32),
                       pltpu.VMEM((CH,), jnp.int32),
                       evmem, evmem, evmem, evmem, evmem, evmem,
                       tvmem, tvmem, tvmem, tvmem],
        compiler_params=_cparams())
    px, py, pz, e0x, e0y, e0z, e1x, e1y, e1z, kk, ak, wi, wj, l = prep(
        x, y, z, vx, vy, vz, fx, fy, fz, wn, cn, i1, j1)

    itk = pl.kernel(
        _iter_body,
        out_type=(node,) * 9 + (edge,),
        mesh=_mesh(),
        scratch_types=[pltpu.VMEM_SHARED((NPAD,), f32),
                       pltpu.VMEM_SHARED((NPAD,), f32),
                       pltpu.VMEM_SHARED((NPAD,), f32),
                       pltpu.VMEM_SHARED((NPAD,), f32),
                       pltpu.VMEM_SHARED((NPAD,), f32),
                       pltpu.VMEM_SHARED((NPAD,), f32),
                       pltpu.VMEM((CH,), jnp.int32),
                       pltpu.VMEM((CH,), jnp.int32),
                       evmem, evmem, evmem, evmem, evmem, evmem,
                       evmem, evmem, evmem, evmem, evmem, evmem,
                       evmem, evmem, evmem, evmem, evmem, evmem,
                       cvmem, cvmem, cvmem],
        compiler_params=_cparams())

    for _ in range(ITERATION):
        px, py, pz, e0x, e0y, e0z, e1x, e1y, e1z, l = itk(
            px, py, pz, e0x, e0y, e0z, e1x, e1y, e1z,
            i1, j1, kk, ak, wi, wj, d0e, l)

    fin = pl.kernel(
        _final_body,
        out_type=(node,) * 6,
        mesh=_mesh(),
        scratch_types=[tvmem, tvmem, tvmem],
        compiler_params=_cparams())
    return fin(px, py, pz, e0x, e0y, e0z, e1x, e1y, e1z, x, y, z)


def kernel(V, V_velocity, V_w, V_force, V_compliance, C_dist, C_init_d):
    f32 = jnp.float32
    n = V.shape[0]
    e = C_dist.shape[0]

    def padn(a):
        return jnp.zeros((NPAD,), f32).at[:n].set(a.astype(f32))

    V = V.astype(f32)
    x = padn(V[:, 0])
    x = x.at[n:].set(jnp.float32(1.0) + jnp.arange(NPAD - n, dtype=f32))
    y = padn(V[:, 1])
    z = padn(V[:, 2])
    vx = padn(V_velocity[:, 0])
    vy = padn(V_velocity[:, 1])
    vz = padn(V_velocity[:, 2])
    fx = padn(V_force[:, 0])
    fy = padn(V_force[:, 1])
    fz = padn(V_force[:, 2])
    wn = padn(V_w[:, 0])
    cn = padn(V_compliance[:, 0])
    i1 = jnp.full((EPAD,), n, jnp.int32).at[:e].set(C_dist[:, 0].astype(jnp.int32))
    j1 = jnp.full((EPAD,), n + 1, jnp.int32).at[:e].set(C_dist[:, 1].astype(jnp.int32))
    d0 = jnp.ones((EPAD,), f32).at[:e].set(C_init_d[:, 0].astype(f32))

    px, py, pz, ux, uy, uz = _xpbd(x, y, z, vx, vy, vz, fx, fy, fz,
                                   wn, cn, i1, j1, d0)
    Vout = jnp.stack([px[:n], py[:n], pz[:n]], axis=1)
    Velout = jnp.stack([ux[:n], uy[:n], uz[:n]], axis=1)
    return Vout, Velout


# speedup vs baseline: 2.5447x; 1.0653x over previous
import jax
import jax.numpy as jnp
from jax import lax
from jax.experimental import pallas as pl
from jax.experimental.pallas import tpu as pltpu
from jax.experimental.pallas import tpu_sc as plsc

N_NODES = 50000
N_EDGES = 1600000
DT = 0.01
ITERATION = 10

NC = 2
NS = 16
NW = NC * NS
NPAD = 50176
ROWS_T = NPAD // NW
ROWS_C = NPAD // NS
CH = 5120
NCH = 10
EPW = NCH * CH
EPAD = EPW * NW

_MAGIC = 0x5F3759DF

_CPARAMS = None


def _cparams():
    global _CPARAMS
    if _CPARAMS is None:
        _CPARAMS = pltpu.CompilerParams(needs_layout_passes=False,
                                        use_tc_tiling_on_sc=False)
    return _CPARAMS


def _mesh():
    return plsc.VectorSubcoreMesh(core_axis_name="c", subcore_axis_name="s",
                                  num_cores=NC)


def _edge_chunks(body_fn):
    @pl.loop(0, NCH)
    def _c(c):
        body_fn(c)


def _prep_body(x_h, y_h, z_h, vx_h, vy_h, vz_h, fx_h, fy_h, fz_h, wn_h, cn_h,
               i1_h, j1_h,
               px_h, py_h, pz_h, e0x_h, e0y_h, e0z_h, e1x_h, e1y_h, e1z_h,
               k_h, ak_h, wi_h, wj_h, l_h,
               ii_v, jj_v, k_v, ak_v, wi_v, wj_v, tmp_v, zc_v,
               buf_a, buf_b, buf_c, buf_w):
    cid = lax.axis_index("c")
    sid = lax.axis_index("s")
    wid = cid * NS + sid
    f32 = jnp.float32
    dt = f32(DT)
    dt2 = f32(DT * DT)
    zero16 = jnp.zeros((16,), f32)

    tsl = pl.ds(wid * ROWS_T, ROWS_T)

    pltpu.sync_copy(wn_h.at[tsl], buf_w)
    for pos_h, vel_h, f_h, p_h in ((x_h, vx_h, fx_h, px_h),
                                   (y_h, vy_h, fy_h, py_h),
                                   (z_h, vz_h, fz_h, pz_h)):
        pltpu.sync_copy(pos_h.at[tsl], buf_a)
        pltpu.sync_copy(vel_h.at[tsl], buf_b)
        pltpu.sync_copy(f_h.at[tsl], buf_c)

        @pl.loop(0, ROWS_T // 16)
        def _pred(t):
            s = pl.ds(t * 16, 16)
            buf_a[s] = buf_a[s] + dt * buf_b[s] + dt2 * buf_w[s] * buf_c[s]

        pltpu.sync_copy(buf_a, p_h.at[tsl])

    @pl.loop(0, ROWS_T // 16)
    def _zb(t):
        buf_a[pl.ds(t * 16, 16)] = zero16

    for d_h in (e0x_h, e0y_h, e0z_h, e1x_h, e1y_h, e1z_h):
        pltpu.sync_copy(buf_a, d_h.at[tsl])

    @pl.loop(0, CH // 16)
    def _zc(t):
        zc_v[pl.ds(t * 16, 16)] = zero16

    @pl.loop(0, NCH)
    def _coef(c):
        sl = pl.ds(wid * EPW + c * CH, CH)
        pltpu.sync_copy(i1_h.at[sl], ii_v)
        pltpu.sync_copy(j1_h.at[sl], jj_v)
        pltpu.sync_copy(wn_h.at[ii_v], wi_v)
        pltpu.sync_copy(wn_h.at[jj_v], wj_v)
        pltpu.sync_copy(cn_h.at[ii_v], k_v)
        pltpu.sync_copy(cn_h.at[jj_v], tmp_v)

        @pl.loop(0, CH // 16)
        def _ck(t):
            s = pl.ds(t * 16, 16)
            wi = wi_v[s]
            wj = wj_v[s]
            a = f32(0.5) * (k_v[s] + tmp_v[s])
            ssum = wi + wj
            k = jnp.where(ssum == 0.0, f32(0.0), f32(1.0) / (ssum + a))
            k_v[s] = k
            ak_v[s] = a * k

        pltpu.sync_copy(k_v, k_h.at[sl])
        pltpu.sync_copy(ak_v, ak_h.at[sl])
        pltpu.sync_copy(wi_v, wi_h.at[sl])
        pltpu.sync_copy(wj_v, wj_h.at[sl])
        pltpu.sync_copy(zc_v, l_h.at[sl])


def _iter_body(pxp_h, pyp_h, pzp_h, d0x_h, d0y_h, d0z_h, d1x_h, d1y_h, d1z_h,
               i1_h, j1_h, k_h, ak_h, wi_h, wj_h, d0e_h, lp_h,
               px_h, py_h, pz_h, e0x_h, e0y_h, e0z_h, e1x_h, e1y_h, e1z_h,
               ln_h,
               ax_sh, ay_sh, az_sh, px_sh, py_sh, pz_sh,
               ii_v, jj_v, k_v, ak_v, wi_v, wj_v, d0_v, l_v,
               gxi, gyi, gzi, gxj, gyj, gzj,
               dxi, dyi, dzi, dxj, dyj, dzj,
               buf_a, buf_b, buf_c):
    cid = lax.axis_index("c")
    sid = lax.axis_index("s")
    wid = cid * NS + sid
    f32 = jnp.float32
    zero16 = jnp.zeros((16,), f32)
    nan16 = zero16 + f32(jnp.nan)

    csl = pl.ds(sid * ROWS_C, ROWS_C)

    for pp_h, da_h, db_h, pn_h, p_sh in ((pxp_h, d0x_h, d1x_h, px_h, px_sh),
                                         (pyp_h, d0y_h, d1y_h, py_h, py_sh),
                                         (pzp_h, d0z_h, d1z_h, pz_h, pz_sh)):
        pltpu.sync_copy(pp_h.at[csl], buf_a)
        pltpu.sync_copy(da_h.at[csl], buf_b)
        pltpu.sync_copy(db_h.at[csl], buf_c)

        @pl.loop(0, ROWS_C // 16)
        def _cmb(t):
            s = pl.ds(t * 16, 16)
            buf_a[s] = buf_a[s] + buf_b[s] + buf_c[s]

        pltpu.sync_copy(buf_a, pn_h.at[csl])
        pltpu.sync_copy(buf_a, p_sh.at[csl])

    @pl.loop(0, ROWS_C // 16)
    def _zb(t):
        buf_b[pl.ds(t * 16, 16)] = zero16

    pltpu.sync_copy(buf_b, ax_sh.at[csl])
    pltpu.sync_copy(buf_b, ay_sh.at[csl])
    pltpu.sync_copy(buf_b, az_sh.at[csl])
    plsc.subcore_barrier()

    @pl.loop(0, NCH)
    def _chunk(c):
        sl = pl.ds(wid * EPW + c * CH, CH)
        pltpu.sync_copy(i1_h.at[sl], ii_v)
        pltpu.sync_copy(j1_h.at[sl], jj_v)
        pltpu.sync_copy(k_h.at[sl], k_v)
        pltpu.sync_copy(ak_h.at[sl], ak_v)
        pltpu.sync_copy(wi_h.at[sl], wi_v)
        pltpu.sync_copy(wj_h.at[sl], wj_v)
        pltpu.sync_copy(d0e_h.at[sl], d0_v)
        pltpu.sync_copy(lp_h.at[sl], l_v)
        pltpu.sync_copy(px_sh.at[ii_v], gxi)
        pltpu.sync_copy(py_sh.at[ii_v], gyi)
        pltpu.sync_copy(pz_sh.at[ii_v], gzi)
        pltpu.sync_copy(px_sh.at[jj_v], gxj)
        pltpu.sync_copy(py_sh.at[jj_v], gyj)
        pltpu.sync_copy(pz_sh.at[jj_v], gzj)

        @pl.loop(0, CH // 16)
        def _edge(t):
            s = pl.ds(t * 16, 16)
            dx = gxi[s] - gxj[s]
            dy = gyi[s] - gyj[s]
            dz = gzi[s] - gzj[s]
            d2 = dx * dx + dy * dy + dz * dz
            bits = plsc.bitcast(d2, jnp.int32)
            y = plsc.bitcast(_MAGIC - (bits >> 1), f32)
            hd = f32(0.5) * d2
            y = y * (f32(1.5) - hd * y * y)
            y = y * (f32(1.5) - hd * y * y)
            dnorm = d2 * y
            invd = jnp.where(d2 == 0.0, nan16, y)
            el = l_v[s]
            ld = (d0_v[s] - dnorm) * k_v[s] - ak_v[s] * el
            l_v[s] = el + ld
            g = ld * invd
            ai = wi_v[s] * g
            aj = -(wj_v[s] * g)
            dxi[s] = ai * dx
            dyi[s] = ai * dy
            dzi[s] = ai * dz
            dxj[s] = aj * dx
            dyj[s] = aj * dy
            dzj[s] = aj * dz

        pltpu.sync_copy(l_v, ln_h.at[sl])
        pltpu.sync_copy(dxi, ax_sh.at[ii_v], add=True)
        pltpu.sync_copy(dyi, ay_sh.at[ii_v], add=True)
        pltpu.sync_copy(dzi, az_sh.at[ii_v], add=True)
        pltpu.sync_copy(dxj, ax_sh.at[jj_v], add=True)
        pltpu.sync_copy(dyj, ay_sh.at[jj_v], add=True)
        pltpu.sync_copy(dzj, az_sh.at[jj_v], add=True)

    plsc.subcore_barrier()

    @pl.when(cid == 0)
    def _pub0():
        for acc_sh, e_h in ((ax_sh, e0x_h), (ay_sh, e0y_h), (az_sh, e0z_h)):
            pltpu.sync_copy(acc_sh.at[csl], buf_a)
            pltpu.sync_copy(buf_a, e_h.at[csl])

    @pl.when(cid == 1)
    def _pub1():
        for acc_sh, e_h in ((ax_sh, e1x_h), (ay_sh, e1y_h), (az_sh, e1z_h)):
            pltpu.sync_copy(acc_sh.at[csl], buf_a)
            pltpu.sync_copy(buf_a, e_h.at[csl])


def _final_body(pxp_h, pyp_h, pzp_h, d0x_h, d0y_h, d0z_h,
                d1x_h, d1y_h, d1z_h, x_h, y_h, z_h,
                px_h, py_h, pz_h, ux_h, uy_h, uz_h,
                buf_a, buf_b, buf_c):
    cid = lax.axis_index("c")
    sid = lax.axis_index("s")
    wid = cid * NS + sid
    f32 = jnp.float32
    dt = f32(DT)
    tsl = pl.ds(wid * ROWS_T, ROWS_T)

    for pp_h, da_h, db_h, o_h, pn_h, u_h in (
            (pxp_h, d0x_h, d1x_h, x_h, px_h, ux_h),
            (pyp_h, d0y_h, d1y_h, y_h, py_h, uy_h),
            (pzp_h, d0z_h, d1z_h, z_h, pz_h, uz_h)):
        pltpu.sync_copy(pp_h.at[tsl], buf_a)
        pltpu.sync_copy(da_h.at[tsl], buf_b)
        pltpu.sync_copy(db_h.at[tsl], buf_c)

        @pl.loop(0, ROWS_T // 16)
        def _cmb(t):
            s = pl.ds(t * 16, 16)
            buf_a[s] = buf_a[s] + buf_b[s] + buf_c[s]

        pltpu.sync_copy(buf_a, pn_h.at[tsl])
        pltpu.sync_copy(o_h.at[tsl], buf_b)

        @pl.loop(0, ROWS_T // 16)
        def _vel(t):
            s = pl.ds(t * 16, 16)
            buf_b[s] = (buf_a[s] - buf_b[s]) / dt

        pltpu.sync_copy(buf_b, u_h.at[tsl])


@jax.jit
def _xpbd(x, y, z, vx, vy, vz, fx, fy, fz, wn, cn, i1, j1, d0e):
    f32 = jnp.float32
    node = jax.ShapeDtypeStruct((NPAD,), f32)
    edge = jax.ShapeDtypeStruct((EPAD,), f32)
    evmem = pltpu.VMEM((CH,), f32)
    tvmem = pltpu.VMEM((ROWS_T,), f32)
    cvmem = pltpu.VMEM((ROWS_C,), f32)

    prep = pl.kernel(
        _prep_body,
        out_type=(node,) * 9 + (edge,) * 5,
        mesh=_mesh(),
        scratch_types=[pltpu.VMEM((CH,), jnp.int32),
                       pltpu.VMEM((CH,), jnp.int32),
                       evmem, evmem, evmem, evmem, evmem, evmem,
                       tvmem, tvmem, tvmem, tvmem],
        compiler_params=_cparams())
    px, py, pz, e0x, e0y, e0z, e1x, e1y, e1z, kk, ak, wi, wj, l = prep(
        x, y, z, vx, vy, vz, fx, fy, fz, wn, cn, i1, j1)

    itk = pl.kernel(
        _iter_body,
        out_type=(node,) * 9 + (edge,),
        mesh=_mesh(),
        scratch_types=[pltpu.VMEM_SHARED((NPAD,), f32),
                       pltpu.VMEM_SHARED((NPAD,), f32),
                       pltpu.VMEM_SHARED((NPAD,), f32),
                       pltpu.VMEM_SHARED((NPAD,), f32),
                       pltpu.VMEM_SHARED((NPAD,), f32),
                       pltpu.VMEM_SHARED((NPAD,), f32),
                       pltpu.VMEM((CH,), jnp.int32),
                       pltpu.VMEM((CH,), jnp.int32),
                       evmem, evmem, evmem, evmem, evmem, evmem,
                       evmem, evmem, evmem, evmem, evmem, evmem,
                       evmem, evmem, evmem, evmem, evmem, evmem,
                       cvmem, cvmem, cvmem],
        compiler_params=_cparams())

    for _ in range(ITERATION):
        px, py, pz, e0x, e0y, e0z, e1x, e1y, e1z, l = itk(
            px, py, pz, e0x, e0y, e0z, e1x, e1y, e1z,
            i1, j1, kk, ak, wi, wj, d0e, l)

    fin = pl.kernel(
        _final_body,
        out_type=(node,) * 6,
        mesh=_mesh(),
        scratch_types=[tvmem, tvmem, tvmem],
        compiler_params=_cparams())
    return fin(px, py, pz, e0x, e0y, e0z, e1x, e1y, e1z, x, y, z)


def kernel(V, V_velocity, V_w, V_force, V_compliance, C_dist, C_init_d):
    f32 = jnp.float32
    n = V.shape[0]
    e = C_dist.shape[0]

    def padn(a):
        return jnp.zeros((NPAD,), f32).at[:n].set(a.astype(f32))

    V = V.astype(f32)
    x = padn(V[:, 0])
    x = x.at[n:].set(jnp.float32(1.0) + jnp.arange(NPAD - n, dtype=f32))
    y = padn(V[:, 1])
    z = padn(V[:, 2])
    vx = padn(V_velocity[:, 0])
    vy = padn(V_velocity[:, 1])
    vz = padn(V_velocity[:, 2])
    fx = padn(V_force[:, 0])
    fy = padn(V_force[:, 1])
    fz = padn(V_force[:, 2])
    wn = padn(V_w[:, 0])
    cn = padn(V_compliance[:, 0])
    i1 = jnp.full((EPAD,), n, jnp.int32).at[:e].set(C_dist[:, 0].astype(jnp.int32))
    j1 = jnp.full((EPAD,), n + 1, jnp.int32).at[:e].set(C_dist[:, 1].astype(jnp.int32))
    d0 = jnp.ones((EPAD,), f32).at[:e].set(C_init_d[:, 0].astype(f32))

    px, py, pz, ux, uy, uz = _xpbd(x, y, z, vx, vy, vz, fx, fy, fz,
                                   wn, cn, i1, j1, d0)
    Vout = jnp.stack([px[:n], py[:n], pz[:n]], axis=1)
    Velout = jnp.stack([ux[:n], uy[:n], uz[:n]], axis=1)
    return Vout, Velout
